# Initial kernel scaffold; baseline (speedup 1.0000x reference)
import jax
import jax.numpy as jnp
from jax.experimental import pallas as pl


def kernel(x, table):
    raise NotImplementedError("write your pallas kernel here")


# speedup vs baseline: 1.7983x; 1.7983x over previous
import functools

import jax
import jax.numpy as jnp
from jax import lax
from jax.experimental import pallas as pl
from jax.experimental.pallas import tpu as pltpu
from jax.experimental.pallas import tpu_sc as plsc

NC = 2
NS = 16
NW = NC * NS

EMB = 64
CHUNK = 512


def _make_lookup(total: int):
    b_per_w = total // NW
    n_chunks = b_per_w // CHUNK
    mesh = plsc.VectorSubcoreMesh(core_axis_name="c", subcore_axis_name="s")

    @functools.partial(
        pl.kernel,
        mesh=mesh,
        compiler_params=pltpu.CompilerParams(use_tc_tiling_on_sc=False),
        out_type=jax.ShapeDtypeStruct((total, EMB), jnp.float32),
        scratch_types=[
            pltpu.VMEM((CHUNK,), jnp.int32),
            pltpu.VMEM((CHUNK, EMB), jnp.float32),
            pltpu.SemaphoreType.DMA,
        ],
    )
    def lookup(x_hbm, table_hbm, out_hbm, idx_v, rows_v, sem):
        wid = lax.axis_index("s") * NC + lax.axis_index("c")
        base = wid * b_per_w

        def body(j, carry):
            pltpu.sync_copy(x_hbm.at[wid, j], idx_v)
            pltpu.async_copy(table_hbm.at[idx_v], rows_v, sem).wait()
            pltpu.sync_copy(rows_v, out_hbm.at[pl.ds(base + j * CHUNK, CHUNK)])
            return carry

        lax.fori_loop(0, n_chunks, body, 0)

    return lookup


def kernel(x, table):
    B, L = x.shape
    total = B * L
    x_flat = x.reshape(NW, total // NW // CHUNK, CHUNK)
    out = _make_lookup(total)(x_flat, table)
    return out.reshape(B, L, EMB)


# speedup vs baseline: 1.8757x; 1.0430x over previous
import functools

import jax
import jax.numpy as jnp
from jax import lax
from jax.experimental import pallas as pl
from jax.experimental.pallas import tpu as pltpu
from jax.experimental.pallas import tpu_sc as plsc

NC = 2
NS = 16
NW = NC * NS

EMB = 64
CHUNK = 256
NBUF = 4


def _make_lookup(total: int):
    b_per_w = total // NW
    n_chunks = b_per_w // CHUNK
    n_super = n_chunks // NBUF
    mesh = plsc.VectorSubcoreMesh(core_axis_name="c", subcore_axis_name="s")

    @functools.partial(
        pl.kernel,
        mesh=mesh,
        compiler_params=pltpu.CompilerParams(use_tc_tiling_on_sc=False),
        out_type=jax.ShapeDtypeStruct((total, EMB), jnp.float32),
        scratch_types=[
            pltpu.VMEM((b_per_w,), jnp.int32),
            [pltpu.VMEM((CHUNK, EMB), jnp.float32) for _ in range(NBUF)],
            [pltpu.SemaphoreType.DMA for _ in range(NBUF)],
            [pltpu.SemaphoreType.DMA for _ in range(NBUF)],
        ],
    )
    def lookup(x_hbm, table_hbm, out_hbm, idx_v, rows, gsem, wsem):
        wid = lax.axis_index("s") * NC + lax.axis_index("c")
        base = wid * b_per_w
        pltpu.sync_copy(x_hbm.at[wid], idx_v)

        def gather(j, b):
            pltpu.async_copy(
                table_hbm.at[idx_v.at[pl.ds(j * CHUNK, CHUNK)]], rows[b], gsem[b]
            )

        def write(j, b):
            pltpu.async_copy(
                rows[b], out_hbm.at[pl.ds(base + j * CHUNK, CHUNK)], wsem[b]
            )

        def gwait(b):
            pltpu.make_async_copy(
                out_hbm.at[pl.ds(base, CHUNK)], rows[b], gsem[b]
            ).wait()

        def wwait(b):
            pltpu.make_async_copy(
                rows[b], out_hbm.at[pl.ds(base, CHUNK)], wsem[b]
            ).wait()

        for b in range(NBUF):
            gather(b, b)

        def body(si, carry):
            for b in range(NBUF):
                j = si * NBUF + b
                gwait(b)
                write(j, b)
                wwait(b)
                gather(j + NBUF, b)
            return carry

        lax.fori_loop(0, n_super - 1, body, 0)

        for b in range(NBUF):
            j = (n_super - 1) * NBUF + b
            gwait(b)
            write(j, b)
        for b in range(NBUF):
            wwait(b)

    return lookup


def kernel(x, table):
    B, L = x.shape
    total = B * L
    x_flat = x.reshape(NW, total // NW)
    out = _make_lookup(total)(x_flat, table)
    return out.reshape(B, L, EMB)


# speedup vs baseline: 2.3176x; 1.2356x over previous
import functools

import jax
import jax.numpy as jnp
from jax import lax
from jax.experimental import pallas as pl
from jax.experimental.pallas import tpu as pltpu
from jax.experimental.pallas import tpu_sc as plsc

NC = 2
NS = 16
NW = NC * NS

EMB = 64
BGRP = 4
NBUF = 4


def _make_lookup(B: int, L: int):
    b_per_w = B // NW
    chunk = BGRP * L
    n_chunks = b_per_w // BGRP
    n_super = n_chunks // NBUF
    mesh = plsc.VectorSubcoreMesh(core_axis_name="c", subcore_axis_name="s")

    @functools.partial(
        pl.kernel,
        mesh=mesh,
        compiler_params=pltpu.CompilerParams(use_tc_tiling_on_sc=True),
        out_type=jax.ShapeDtypeStruct((B, L, 128), jnp.float32),
        scratch_types=[
            pltpu.VMEM((b_per_w * L,), jnp.int32),
            [pltpu.VMEM((chunk, 128), jnp.float32) for _ in range(NBUF)],
            [pltpu.SemaphoreType.DMA for _ in range(NBUF)],
            [pltpu.SemaphoreType.DMA for _ in range(NBUF)],
        ],
    )
    def lookup(x_hbm, table_hbm, out_hbm, idx_v, rows, gsem, wsem):
        wid = lax.axis_index("s") * NC + lax.axis_index("c")
        base_b = wid * b_per_w
        pltpu.sync_copy(x_hbm.at[wid], idx_v)

        def gather(j, b):
            pltpu.async_copy(
                table_hbm.at[idx_v.at[pl.ds(j * chunk, chunk)]], rows[b], gsem[b]
            )

        def write(j, b):
            for g in range(BGRP):
                pltpu.async_copy(
                    rows[b].at[pl.ds(g * L, L), :],
                    out_hbm.at[base_b + j * BGRP + g],
                    wsem[b],
                )

        def gwait(b):
            pltpu.make_async_copy(
                table_hbm.at[pl.ds(0, chunk)], rows[b], gsem[b]
            ).wait()

        def wwait(b):
            for g in range(BGRP):
                pltpu.make_async_copy(
                    rows[b].at[pl.ds(g * L, L), :],
                    out_hbm.at[base_b + g],
                    wsem[b],
                ).wait()

        for b in range(NBUF):
            gather(b, b)

        def body(si, carry):
            for b in range(NBUF):
                j = si * NBUF + b
                gwait(b)
                write(j, b)
                wwait(b)
                gather(j + NBUF, b)
            return carry

        lax.fori_loop(0, n_super - 1, body, 0)

        for b in range(NBUF):
            j = (n_super - 1) * NBUF + b
            gwait(b)
            write(j, b)
        for b in range(NBUF):
            wwait(b)

    return lookup


def kernel(x, table):
    B, L = x.shape
    x_flat = x.reshape(NW, B // NW * L)
    table_fat = jnp.pad(table, ((0, 0), (0, 128 - EMB)))
    out_fat = _make_lookup(B, L)(x_flat, table_fat)
    return out_fat[:, :, :EMB]
